# Initial kernel scaffold; baseline (speedup 1.0000x reference)
import jax
import jax.numpy as jnp
from jax.experimental import pallas as pl


def kernel(z_e, codebook):
    raise NotImplementedError("write your pallas kernel here")


# speedup vs baseline: 3.1602x; 3.1602x over previous
import jax
import jax.numpy as jnp
from jax.experimental import pallas as pl

NUM_CODES = 512
CODE_DIM = 32
TOK_BLK = 512


def _vq_kernel(z_ref, cb_ref, zq_ref, idx_ref):
    x = z_ref[:]
    c = cb_ref[:]
    cnorm2 = jnp.sum(c * c, axis=1, keepdims=True)
    scores_t = jax.lax.dot_general(
        c, x, (((1,), (1,)), ((), ())),
        precision=jax.lax.Precision.HIGHEST,
        preferred_element_type=jnp.float32)
    dist2 = cnorm2 - 2.0 * scores_t
    m = jnp.min(dist2, axis=0, keepdims=True)
    sub = jax.lax.broadcasted_iota(jnp.int32, (NUM_CODES, TOK_BLK), 0)
    idx = jnp.min(jnp.where(dist2 == m, sub, NUM_CODES),
                  axis=0, keepdims=True)
    onehot = (sub == idx).astype(jnp.float32)
    zq_ref[:] = jax.lax.dot_general(
        onehot, c, (((0,), (0,)), ((), ())),
        precision=jax.lax.Precision.HIGHEST,
        preferred_element_type=jnp.float32)
    idx_ref[0] = idx


def kernel(z_e, codebook):
    B, S, D = z_e.shape
    tok = B * S
    nblk = tok // TOK_BLK
    z2 = z_e.reshape(tok, D)
    zq, idx = pl.pallas_call(
        _vq_kernel,
        grid=(nblk,),
        in_specs=[
            pl.BlockSpec((TOK_BLK, D), lambda i: (i, 0)),
            pl.BlockSpec((NUM_CODES, D), lambda i: (0, 0)),
        ],
        out_specs=[
            pl.BlockSpec((TOK_BLK, D), lambda i: (i, 0)),
            pl.BlockSpec((1, 1, TOK_BLK), lambda i: (i, 0, 0)),
        ],
        out_shape=[
            jax.ShapeDtypeStruct((tok, D), jnp.float32),
            jax.ShapeDtypeStruct((nblk, 1, TOK_BLK), jnp.int32),
        ],
    )(z2, codebook)
    return zq.reshape(B, S, D), idx.reshape(B, S)


# speedup vs baseline: 5.9018x; 1.8675x over previous
import jax
import jax.numpy as jnp
from jax.experimental import pallas as pl

NUM_CODES = 512
CODE_DIM = 32
TOK_BLK = 4096


def _vq_kernel(z_ref, cb_ref, zq_ref, idx_ref):
    x = z_ref[:]
    c = cb_ref[:]
    cnorm2 = jnp.sum(c * c, axis=1, keepdims=True)
    scores_t = jax.lax.dot_general(
        c, x, (((1,), (1,)), ((), ())),
        precision=jax.lax.Precision.HIGHEST,
        preferred_element_type=jnp.float32)
    dist2 = cnorm2 - 2.0 * scores_t
    m = jnp.min(dist2, axis=0, keepdims=True)
    sub = jax.lax.broadcasted_iota(jnp.int32, (NUM_CODES, TOK_BLK), 0)
    idx = jnp.min(jnp.where(dist2 == m, sub, NUM_CODES),
                  axis=0, keepdims=True)
    onehot = (sub == idx).astype(jnp.float32)
    zq_ref[:] = jax.lax.dot_general(
        onehot, c, (((0,), (0,)), ((), ())),
        preferred_element_type=jnp.float32)
    idx_ref[0] = idx


def kernel(z_e, codebook):
    B, S, D = z_e.shape
    tok = B * S
    nblk = tok // TOK_BLK
    z2 = z_e.reshape(tok, D)
    zq, idx = pl.pallas_call(
        _vq_kernel,
        grid=(nblk,),
        in_specs=[
            pl.BlockSpec((TOK_BLK, D), lambda i: (i, 0)),
            pl.BlockSpec((NUM_CODES, D), lambda i: (0, 0)),
        ],
        out_specs=[
            pl.BlockSpec((TOK_BLK, D), lambda i: (i, 0)),
            pl.BlockSpec((1, 1, TOK_BLK), lambda i: (i, 0, 0)),
        ],
        out_shape=[
            jax.ShapeDtypeStruct((tok, D), jnp.float32),
            jax.ShapeDtypeStruct((nblk, 1, TOK_BLK), jnp.int32),
        ],
    )(z2, codebook)
    return zq.reshape(B, S, D), idx.reshape(B, S)
